# baseline (speedup 1.0000x reference)
import functools

import jax
import jax.numpy as jnp
from jax import lax
from jax.experimental import pallas as pl
from jax.experimental.pallas import tpu as pltpu
from jax.experimental.pallas import tpu_sc as plsc

NUM_CORES = 2
NUM_SUBCORES = 16
NUM_WORKERS = NUM_CORES * NUM_SUBCORES
CHUNK = 128
NBUF = 8
DIM = 64
LANES = 16
SCALE = 8.0


@functools.lru_cache(maxsize=None)
def _make_lookup(n_chunks: int):
    mesh = plsc.VectorSubcoreMesh(core_axis_name="c", subcore_axis_name="s")

    @functools.partial(
        pl.kernel,
        mesh=mesh,
        out_type=jax.ShapeDtypeStruct((NUM_WORKERS, n_chunks, CHUNK, DIM),
                                      jnp.float32),
        scratch_types=(
            [pltpu.VMEM((n_chunks, CHUNK), jnp.int32)]
            + [pltpu.VMEM((CHUNK, DIM), jnp.float32)] * (NBUF + 2)
            + [pltpu.SemaphoreType.DMA] * (NBUF + 2)
        ),
        compiler_params=pltpu.CompilerParams(use_tc_tiling_on_sc=False),
    )
    def lookup(table_hbm, idx_hbm, out_hbm, idx_v, *bufs_sems):
        gbufs = bufs_sems[:NBUF]
        sbufs = bufs_sems[NBUF:NBUF + 2]
        gsems = bufs_sems[NBUF + 2:2 * NBUF + 2]
        ssems = bufs_sems[2 * NBUF + 2:]
        wid = lax.axis_index("s") * NUM_CORES + lax.axis_index("c")

        pltpu.sync_copy(idx_hbm.at[wid], idx_v)

        def issue_gather(t):
            return pltpu.async_copy(
                table_hbm.at[idx_v.at[t]], gbufs[t % NBUF], gsems[t % NBUF])

        def issue_store(t):
            return pltpu.async_copy(
                sbufs[t % 2], out_hbm.at[wid, t], ssems[t % 2])

        def scale_chunk(t):
            src = gbufs[t % NBUF]
            dst = sbufs[t % 2]

            @plsc.parallel_loop(0, CHUNK, step=1, unroll=8)
            def _row(r):
                for cc in range(DIM // LANES):
                    sl = pl.ds(cc * LANES, LANES)
                    dst[r, sl] = src[r, sl] * jnp.float32(SCALE)

        gathers = {}
        for t in range(min(NBUF, n_chunks)):
            gathers[t] = issue_gather(t)
        stores = {}
        for t in range(n_chunks):
            gathers[t].wait()
            if t >= 2:
                stores[t - 2].wait()
            scale_chunk(t)
            stores[t] = issue_store(t)
            if t + NBUF < n_chunks:
                gathers[t + NBUF] = issue_gather(t + NBUF)
        stores[n_chunks - 2].wait()
        stores[n_chunks - 1].wait()

    return lookup


def kernel(inputSWE, table):
    s, n = inputSWE.shape
    total = s * n
    n_chunks = total // (NUM_WORKERS * CHUNK)
    idx = inputSWE.reshape(NUM_WORKERS, n_chunks, CHUNK).astype(jnp.int32)
    out = _make_lookup(n_chunks)(table, idx)
    return out.reshape(s, n, DIM)
